# baseline (speedup 1.0000x reference)
import functools

import jax
import jax.numpy as jnp
from jax import lax
from jax.experimental import pallas as pl
from jax.experimental.pallas import tpu as pltpu
from jax.experimental.pallas import tpu_sc as plsc

_NC = 2
_NS = 16
_CH = 80
_IG = 16
_F0 = 0.76


def _mesh():
    return plsc.VectorSubcoreMesh(
        core_axis_name="c", subcore_axis_name="s",
        num_cores=_NC, num_subcores=_NS)


def _acc_rows(n):
    per_tile = -(-(n + 1) // _NS)
    rpt = -(-per_tile // _CH) * _CH
    return rpt, rpt * _NS


def _split_edges(src, dst, n, e):
    ntot = -(-e // (_NS * _CH * _IG)) * _IG
    nch0 = int(round(_F0 * ntot / _IG)) * _IG
    nch0 = min(max(nch0, _IG), ntot - _IG)
    nch1 = ntot - nch0
    ep = _NS * ntot * _CH
    pad = ep - e
    srcp = jnp.concatenate([src, jnp.zeros((pad,), src.dtype)])
    dstp = jnp.concatenate([dst, jnp.full((pad,), n, dst.dtype)])
    return (srcp.reshape(_NS * ntot, _CH), dstp.reshape(_NS * ntot, _CH),
            nch0, nch1)


def _core_slab(c, s, nch0, nch1):
    base = jnp.where(c == 0, s * nch0, _NS * nch0 + s * nch1)
    ngrp = jnp.where(c == 0, nch0 // _IG, nch1 // _IG)
    return base, ngrp


def _deg_partials(dstf, n, nch0, nch1):
    rpt, np_ = _acc_rows(n)

    @functools.partial(
        pl.kernel,
        out_type=jax.ShapeDtypeStruct((_NC * np_, 16), jnp.float32),
        mesh=_mesh(),
        scratch_types=[
            pltpu.VMEM((_IG, _CH), jnp.int32),
            pltpu.VMEM((_CH, 16), jnp.float32),
            pltpu.VMEM_SHARED((np_, 16), jnp.float32),
        ],
    )
    def deg_k(dstf_hbm, out_hbm, dst_v, buf_v, acc):
        c = lax.axis_index("c")
        s = lax.axis_index("s")
        base, ngrp = _core_slab(c, s, nch0, nch1)
        row0 = s * rpt

        def fill(val16):
            def fb(r, _):
                buf_v[r, :] = val16
                return 0
            lax.fori_loop(0, _CH, fb, 0)

        fill(jnp.zeros((16,), jnp.float32))

        def zout(j, _):
            pltpu.sync_copy(buf_v, acc.at[pl.ds(row0 + j * _CH, _CH)])
            return 0
        lax.fori_loop(0, rpt // _CH, zout, 0)

        fill(jnp.ones((16,), jnp.float32))
        plsc.subcore_barrier()

        def group(g, _):
            @pl.when(g < ngrp)
            def _():
                pltpu.sync_copy(dstf_hbm.at[pl.ds(base + g * _IG, _IG)], dst_v)

                def body(k, _):
                    pltpu.sync_copy(buf_v, acc.at[dst_v.at[k]], add=True)
                    return 0
                lax.fori_loop(0, _IG, body, 0)
            return 0
        lax.fori_loop(0, max(nch0, nch1) // _IG, group, 0)

        plsc.subcore_barrier()

        def cout(j, _):
            pltpu.sync_copy(acc.at[pl.ds(row0 + j * _CH, _CH)], buf_v)
            pltpu.sync_copy(
                buf_v, out_hbm.at[pl.ds(c * np_ + row0 + j * _CH, _CH)])
            return 0
        lax.fori_loop(0, rpt // _CH, cout, 0)

    return deg_k(dstf).reshape(_NC, np_, 16)


def _aggregate(srcf, dstf, hs, n, nch0, nch1, d):
    rpt, np_ = _acc_rows(n)

    @functools.partial(
        pl.kernel,
        out_type=jax.ShapeDtypeStruct((_NC * np_, d), jnp.float32),
        mesh=_mesh(),
        scratch_types=[
            pltpu.VMEM((_IG, _CH), jnp.int32),
            pltpu.VMEM((_IG, _CH), jnp.int32),
            pltpu.VMEM((2, _CH, d), jnp.float32),
            pltpu.VMEM_SHARED((np_, d), jnp.float32),
            pltpu.SemaphoreType.DMA,
            pltpu.SemaphoreType.DMA,
        ],
    )
    def agg_k(srcf_hbm, dstf_hbm, hs_hbm, out_hbm,
              src_v, dst_v, rows_v, acc, sem0, sem1):
        gsems = (sem0, sem1)
        c = lax.axis_index("c")
        s = lax.axis_index("s")
        base, ngrp = _core_slab(c, s, nch0, nch1)
        row0 = s * rpt
        z16 = jnp.zeros((16,), jnp.float32)

        with jax.named_scope("agg_zero"):
            def zrow(r, _):
                def zcol(k, _):
                    rows_v[0, r, pl.ds(k * 16, 16)] = z16
                    return 0
                lax.fori_loop(0, d // 16, zcol, 0)
                return 0
            lax.fori_loop(0, _CH, zrow, 0)

            def zout(j, _):
                pltpu.sync_copy(rows_v.at[0], acc.at[pl.ds(row0 + j * _CH, _CH)])
                return 0
            lax.fori_loop(0, rpt // _CH, zout, 0)

            plsc.subcore_barrier()

        def fire_gather(k, b):
            pltpu.async_copy(hs_hbm.at[src_v.at[k]], rows_v.at[b], gsems[b])

        def wait_gather(k, b):
            pltpu.make_async_copy(
                hs_hbm.at[src_v.at[k]], rows_v.at[b], gsems[b]).wait()

        def scatter(k, b):
            pltpu.sync_copy(rows_v.at[b], acc.at[dst_v.at[k]], add=True)

        def group(g, _):
            @pl.when(g < ngrp)
            def _():
                pltpu.sync_copy(srcf_hbm.at[pl.ds(base + g * _IG, _IG)], src_v)
                pltpu.sync_copy(dstf_hbm.at[pl.ds(base + g * _IG, _IG)], dst_v)
                for b in range(2):
                    fire_gather(b, b)

                def inner(o, _):
                    for b in range(2):
                        k = o * 2 + b
                        wait_gather(k, b)
                        scatter(k, b)
                        fire_gather(k + 2, b)
                    return 0
                lax.fori_loop(0, _IG // 2 - 1, inner, 0)

                for b in range(2):
                    k = _IG - 2 + b
                    wait_gather(k, b)
                    scatter(k, b)
            return 0
        with jax.named_scope("agg_loop"):
            lax.fori_loop(0, max(nch0, nch1) // _IG, group, 0)

        with jax.named_scope("agg_bar2"):
            plsc.subcore_barrier()

        with jax.named_scope("agg_cout"):
            def cout(j, _):
                pltpu.sync_copy(acc.at[pl.ds(row0 + j * _CH, _CH)], rows_v.at[0])
                pltpu.sync_copy(
                    rows_v.at[0], out_hbm.at[pl.ds(c * np_ + row0 + j * _CH, _CH)])
                return 0
            lax.fori_loop(0, rpt // _CH, cout, 0)

    return agg_k(srcf, dstf, hs).reshape(_NC, np_, d)


_R = 2000


def _tc_first(degp, x, w1):
    n, din = x.shape
    dh = w1.shape[1]

    def body(deg_ref, x_ref, w_ref, dis_ref, hs_ref):
        deg = deg_ref[0, :, 0:1] + deg_ref[1, :, 0:1] + 1.0
        dis = lax.rsqrt(deg)
        dis_ref[...] = dis
        hs_ref[...] = dis * jnp.dot(x_ref[...], w_ref[...],
                                    preferred_element_type=jnp.float32)

    return pl.pallas_call(
        body,
        grid=(n // _R,),
        in_specs=[
            pl.BlockSpec((_NC, _R, 16), lambda i: (0, i, 0)),
            pl.BlockSpec((_R, din), lambda i: (i, 0)),
            pl.BlockSpec((din, dh), lambda i: (0, 0)),
        ],
        out_specs=[
            pl.BlockSpec((_R, 1), lambda i: (i, 0)),
            pl.BlockSpec((_R, dh), lambda i: (i, 0)),
        ],
        out_shape=[
            jax.ShapeDtypeStruct((n, 1), jnp.float32),
            jax.ShapeDtypeStruct((n, dh), jnp.float32),
        ],
    )(degp, x, w1)


def _tc_mid(p, hs, dis, b, w):
    n, d = hs.shape
    dn = w.shape[1]

    def body(p_ref, hs_ref, dis_ref, b_ref, w_ref, out_ref):
        a = p_ref[0] + p_ref[1] + hs_ref[...]
        h = jnp.maximum(dis_ref[...] * a + b_ref[...], 0.0)
        out_ref[...] = dis_ref[...] * jnp.dot(h, w_ref[...],
                                              preferred_element_type=jnp.float32)

    return pl.pallas_call(
        body,
        grid=(n // _R,),
        in_specs=[
            pl.BlockSpec((_NC, _R, d), lambda i: (0, i, 0)),
            pl.BlockSpec((_R, d), lambda i: (i, 0)),
            pl.BlockSpec((_R, 1), lambda i: (i, 0)),
            pl.BlockSpec((1, d), lambda i: (0, 0)),
            pl.BlockSpec((d, dn), lambda i: (0, 0)),
        ],
        out_specs=pl.BlockSpec((_R, dn), lambda i: (i, 0)),
        out_shape=jax.ShapeDtypeStruct((n, dn), jnp.float32),
    )(p, hs, dis, b, w)


def _tc_last(p, hs, dis, b3, w4, b4):
    n, d = hs.shape
    dn = w4.shape[1]
    dv = w4.shape[0]

    def body(p_ref, hs_ref, dis_ref, b3_ref, w_ref, b4_ref, out_ref):
        a = p_ref[0] + p_ref[1] + hs_ref[...]
        h = (dis_ref[...] * a)[:, :dv] + b3_ref[...]
        out_ref[...] = jax.nn.sigmoid(
            jnp.dot(h, w_ref[...], preferred_element_type=jnp.float32)
            + b4_ref[...])

    return pl.pallas_call(
        body,
        grid=(n // _R,),
        in_specs=[
            pl.BlockSpec((_NC, _R, d), lambda i: (0, i, 0)),
            pl.BlockSpec((_R, d), lambda i: (i, 0)),
            pl.BlockSpec((_R, 1), lambda i: (i, 0)),
            pl.BlockSpec((1, dv), lambda i: (0, 0)),
            pl.BlockSpec((dv, dn), lambda i: (0, 0)),
            pl.BlockSpec((1, dn), lambda i: (0, 0)),
        ],
        out_specs=pl.BlockSpec((_R, dn), lambda i: (i, 0)),
        out_shape=jax.ShapeDtypeStruct((n, dn), jnp.float32),
    )(p, hs, dis, b3, w4, b4)


def kernel(node_features, edge_index, W1, b1, W2, b2, W3, b3, W4, b4):
    x = node_features
    n, _ = x.shape
    e = edge_index.shape[1]
    dh = W1.shape[1]
    do = W3.shape[1]

    srcf, dstf, nch0, nch1 = _split_edges(edge_index[0], edge_index[1], n, e)

    degp = _deg_partials(dstf, n, nch0, nch1)
    dis, hs = _tc_first(degp, x, W1)

    p = _aggregate(srcf, dstf, hs, n, nch0, nch1, dh)
    hs = _tc_mid(p, hs, dis, b1.reshape(1, -1), W2)
    for _ in range(3):
        p = _aggregate(srcf, dstf, hs, n, nch0, nch1, dh)
        hs = _tc_mid(p, hs, dis, b2.reshape(1, -1), W2)

    w3p = jnp.pad(W3, ((0, 0), (0, dh - do)))
    p = _aggregate(srcf, dstf, hs, n, nch0, nch1, dh)
    hs = _tc_mid(p, hs, dis, b2.reshape(1, -1), w3p)

    p = _aggregate(srcf, dstf, hs, n, nch0, nch1, dh)
    return _tc_last(p, hs, dis, b3.reshape(1, -1), W4, b4.reshape(1, -1))


# speedup vs baseline: 2.7807x; 2.7807x over previous
import functools

import jax
import jax.numpy as jnp
from jax import lax
from jax.experimental import pallas as pl
from jax.experimental.pallas import tpu as pltpu
from jax.experimental.pallas import tpu_sc as plsc

_NC = 2
_NS = 16
_CH = 80
_IG = 16
_F0 = 0.5


def _mesh():
    return plsc.VectorSubcoreMesh(
        core_axis_name="c", subcore_axis_name="s",
        num_cores=_NC, num_subcores=_NS)


def _acc_rows(n):
    per_tile = -(-(n + 1) // _NS)
    rpt = -(-per_tile // _CH) * _CH
    return rpt, rpt * _NS


def _split_edges(src, dst, n, e):
    ntot = -(-e // (_NS * _CH * _IG)) * _IG
    nch0 = int(round(_F0 * ntot / _IG)) * _IG
    nch0 = min(max(nch0, _IG), ntot - _IG)
    nch1 = ntot - nch0
    ep = _NS * ntot * _CH
    pad = ep - e
    _, np_ = _acc_rows(n)
    ar = jnp.arange(pad, dtype=src.dtype)
    srcp = jnp.concatenate([src, ar % jnp.asarray(n, src.dtype)])
    dstp = jnp.concatenate(
        [dst, jnp.asarray(n, dst.dtype) + ar % jnp.asarray(np_ - n, dst.dtype)])
    return (srcp.reshape(_NS * ntot, _CH), dstp.reshape(_NS * ntot, _CH),
            nch0, nch1)


def _core_slab(c, s, nch0, nch1):
    base = jnp.where(c == 0, s * nch0, _NS * nch0 + s * nch1)
    ngrp = jnp.where(c == 0, nch0 // _IG, nch1 // _IG)
    return base, ngrp


def _deg_partials(dstf, n, nch0, nch1):
    rpt, np_ = _acc_rows(n)

    @functools.partial(
        pl.kernel,
        out_type=jax.ShapeDtypeStruct((_NC * np_, 16), jnp.float32),
        mesh=_mesh(),
        scratch_types=[
            pltpu.VMEM((_IG, _CH), jnp.int32),
            pltpu.VMEM((_CH, 16), jnp.float32),
            pltpu.VMEM_SHARED((np_, 16), jnp.float32),
        ],
    )
    def deg_k(dstf_hbm, out_hbm, dst_v, buf_v, acc):
        c = lax.axis_index("c")
        s = lax.axis_index("s")
        base, ngrp = _core_slab(c, s, nch0, nch1)
        row0 = s * rpt

        def fill(val16):
            def fb(r, _):
                buf_v[r, :] = val16
                return 0
            lax.fori_loop(0, _CH, fb, 0)

        fill(jnp.zeros((16,), jnp.float32))

        def zout(j, _):
            pltpu.sync_copy(buf_v, acc.at[pl.ds(row0 + j * _CH, _CH)])
            return 0
        lax.fori_loop(0, rpt // _CH, zout, 0)

        fill(jnp.ones((16,), jnp.float32))
        plsc.subcore_barrier()

        def group(g, _):
            @pl.when(g < ngrp)
            def _():
                pltpu.sync_copy(dstf_hbm.at[pl.ds(base + g * _IG, _IG)], dst_v)

                def body(k, _):
                    pltpu.sync_copy(buf_v, acc.at[dst_v.at[k]], add=True)
                    return 0
                lax.fori_loop(0, _IG, body, 0)
            return 0
        lax.fori_loop(0, max(nch0, nch1) // _IG, group, 0)

        plsc.subcore_barrier()

        def cout(j, _):
            pltpu.sync_copy(acc.at[pl.ds(row0 + j * _CH, _CH)], buf_v)
            pltpu.sync_copy(
                buf_v, out_hbm.at[pl.ds(c * np_ + row0 + j * _CH, _CH)])
            return 0
        lax.fori_loop(0, rpt // _CH, cout, 0)

    return deg_k(dstf).reshape(_NC, np_, 16)


def _aggregate(srcf, dstf, hs, n, nch0, nch1, d):
    rpt, np_ = _acc_rows(n)

    @functools.partial(
        pl.kernel,
        out_type=jax.ShapeDtypeStruct((_NC * np_, d), jnp.float32),
        mesh=_mesh(),
        scratch_types=[
            pltpu.VMEM((_IG, _CH), jnp.int32),
            pltpu.VMEM((_IG, _CH), jnp.int32),
            pltpu.VMEM((2, _CH, d), jnp.float32),
            pltpu.VMEM_SHARED((np_, d), jnp.float32),
            pltpu.SemaphoreType.DMA,
            pltpu.SemaphoreType.DMA,
        ],
    )
    def agg_k(srcf_hbm, dstf_hbm, hs_hbm, out_hbm,
              src_v, dst_v, rows_v, acc, sem0, sem1):
        gsems = (sem0, sem1)
        c = lax.axis_index("c")
        s = lax.axis_index("s")
        base, ngrp = _core_slab(c, s, nch0, nch1)
        row0 = s * rpt
        z16 = jnp.zeros((16,), jnp.float32)

        with jax.named_scope("agg_zero"):
            def zrow(r, _):
                def zcol(k, _):
                    rows_v[0, r, pl.ds(k * 16, 16)] = z16
                    return 0
                lax.fori_loop(0, d // 16, zcol, 0)
                return 0
            lax.fori_loop(0, _CH, zrow, 0)

            def zout(j, _):
                pltpu.sync_copy(rows_v.at[0], acc.at[pl.ds(row0 + j * _CH, _CH)])
                return 0
            lax.fori_loop(0, rpt // _CH, zout, 0)

            plsc.subcore_barrier()

        def fire_gather(k, b):
            pltpu.async_copy(hs_hbm.at[src_v.at[k]], rows_v.at[b], gsems[b])

        def wait_gather(k, b):
            pltpu.make_async_copy(
                hs_hbm.at[src_v.at[k]], rows_v.at[b], gsems[b]).wait()

        def scatter(k, b):
            pltpu.sync_copy(rows_v.at[b], acc.at[dst_v.at[k]], add=True)

        def group(g, _):
            @pl.when(g < ngrp)
            def _():
                pltpu.sync_copy(srcf_hbm.at[pl.ds(base + g * _IG, _IG)], src_v)
                pltpu.sync_copy(dstf_hbm.at[pl.ds(base + g * _IG, _IG)], dst_v)
                for b in range(2):
                    fire_gather(b, b)

                def inner(o, _):
                    for b in range(2):
                        k = o * 2 + b
                        wait_gather(k, b)
                        scatter(k, b)
                        fire_gather(k + 2, b)
                    return 0
                lax.fori_loop(0, _IG // 2 - 1, inner, 0)

                for b in range(2):
                    k = _IG - 2 + b
                    wait_gather(k, b)
                    scatter(k, b)
            return 0
        with jax.named_scope("agg_loop"):
            lax.fori_loop(0, max(nch0, nch1) // _IG, group, 0)

        with jax.named_scope("agg_bar2"):
            plsc.subcore_barrier()

        with jax.named_scope("agg_cout"):
            def cout(j, _):
                pltpu.sync_copy(acc.at[pl.ds(row0 + j * _CH, _CH)], rows_v.at[0])
                pltpu.sync_copy(
                    rows_v.at[0], out_hbm.at[pl.ds(c * np_ + row0 + j * _CH, _CH)])
                return 0
            lax.fori_loop(0, rpt // _CH, cout, 0)

    return agg_k(srcf, dstf, hs).reshape(_NC, np_, d)


_R = 2000


def _tc_first(degp, x, w1):
    n, din = x.shape
    dh = w1.shape[1]

    def body(deg_ref, x_ref, w_ref, dis_ref, hs_ref):
        deg = deg_ref[0, :, 0:1] + deg_ref[1, :, 0:1] + 1.0
        dis = lax.rsqrt(deg)
        dis_ref[...] = dis
        hs_ref[...] = dis * jnp.dot(x_ref[...], w_ref[...],
                                    preferred_element_type=jnp.float32)

    return pl.pallas_call(
        body,
        grid=(n // _R,),
        in_specs=[
            pl.BlockSpec((_NC, _R, 16), lambda i: (0, i, 0)),
            pl.BlockSpec((_R, din), lambda i: (i, 0)),
            pl.BlockSpec((din, dh), lambda i: (0, 0)),
        ],
        out_specs=[
            pl.BlockSpec((_R, 1), lambda i: (i, 0)),
            pl.BlockSpec((_R, dh), lambda i: (i, 0)),
        ],
        out_shape=[
            jax.ShapeDtypeStruct((n, 1), jnp.float32),
            jax.ShapeDtypeStruct((n, dh), jnp.float32),
        ],
    )(degp, x, w1)


def _tc_mid(p, hs, dis, b, w):
    n, d = hs.shape
    dn = w.shape[1]

    def body(p_ref, hs_ref, dis_ref, b_ref, w_ref, out_ref):
        a = p_ref[0] + p_ref[1] + hs_ref[...]
        h = jnp.maximum(dis_ref[...] * a + b_ref[...], 0.0)
        out_ref[...] = dis_ref[...] * jnp.dot(h, w_ref[...],
                                              preferred_element_type=jnp.float32)

    return pl.pallas_call(
        body,
        grid=(n // _R,),
        in_specs=[
            pl.BlockSpec((_NC, _R, d), lambda i: (0, i, 0)),
            pl.BlockSpec((_R, d), lambda i: (i, 0)),
            pl.BlockSpec((_R, 1), lambda i: (i, 0)),
            pl.BlockSpec((1, d), lambda i: (0, 0)),
            pl.BlockSpec((d, dn), lambda i: (0, 0)),
        ],
        out_specs=pl.BlockSpec((_R, dn), lambda i: (i, 0)),
        out_shape=jax.ShapeDtypeStruct((n, dn), jnp.float32),
    )(p, hs, dis, b, w)


def _tc_last(p, hs, dis, b3, w4, b4):
    n, d = hs.shape
    dn = w4.shape[1]
    dv = w4.shape[0]

    def body(p_ref, hs_ref, dis_ref, b3_ref, w_ref, b4_ref, out_ref):
        a = p_ref[0] + p_ref[1] + hs_ref[...]
        h = (dis_ref[...] * a)[:, :dv] + b3_ref[...]
        out_ref[...] = jax.nn.sigmoid(
            jnp.dot(h, w_ref[...], preferred_element_type=jnp.float32)
            + b4_ref[...])

    return pl.pallas_call(
        body,
        grid=(n // _R,),
        in_specs=[
            pl.BlockSpec((_NC, _R, d), lambda i: (0, i, 0)),
            pl.BlockSpec((_R, d), lambda i: (i, 0)),
            pl.BlockSpec((_R, 1), lambda i: (i, 0)),
            pl.BlockSpec((1, dv), lambda i: (0, 0)),
            pl.BlockSpec((dv, dn), lambda i: (0, 0)),
            pl.BlockSpec((1, dn), lambda i: (0, 0)),
        ],
        out_specs=pl.BlockSpec((_R, dn), lambda i: (i, 0)),
        out_shape=jax.ShapeDtypeStruct((n, dn), jnp.float32),
    )(p, hs, dis, b3, w4, b4)


def kernel(node_features, edge_index, W1, b1, W2, b2, W3, b3, W4, b4):
    x = node_features
    n, _ = x.shape
    e = edge_index.shape[1]
    dh = W1.shape[1]
    do = W3.shape[1]

    srcf, dstf, nch0, nch1 = _split_edges(edge_index[0], edge_index[1], n, e)

    degp = _deg_partials(dstf, n, nch0, nch1)
    dis, hs = _tc_first(degp, x, W1)

    p = _aggregate(srcf, dstf, hs, n, nch0, nch1, dh)
    hs = _tc_mid(p, hs, dis, b1.reshape(1, -1), W2)
    for _ in range(3):
        p = _aggregate(srcf, dstf, hs, n, nch0, nch1, dh)
        hs = _tc_mid(p, hs, dis, b2.reshape(1, -1), W2)

    w3p = jnp.pad(W3, ((0, 0), (0, dh - do)))
    p = _aggregate(srcf, dstf, hs, n, nch0, nch1, dh)
    hs = _tc_mid(p, hs, dis, b2.reshape(1, -1), w3p)

    p = _aggregate(srcf, dstf, hs, n, nch0, nch1, dh)
    return _tc_last(p, hs, dis, b3.reshape(1, -1), W4, b4.reshape(1, -1))
